# baseline (speedup 1.0000x reference)
import functools

import jax
import jax.numpy as jnp
from jax import lax
from jax.experimental import pallas as pl
from jax.experimental.pallas import tpu as pltpu
from jax.experimental.pallas import tpu_sc as plsc

_K = 5
_SPACE = 3
_BIG = float(1e10)
_HUGE = float(3e38)
_IBIG = 2**31 - 1

_RK = 256
_CT = 512
_RD = 512
_CH = 80
_NW = 32


def _knn(tlo, thi, pos_r, posT, batch_r, batch_c, Np):
    nb = Np // _RK

    def body(tlo_ref, thi_ref, posr_ref, posT_ref, br_ref, bc_ref, out_ref):
        i = pl.program_id(0)
        rpos = posr_ref[...]
        rsq = jnp.sum(rpos * rpos, axis=1, keepdims=True)
        rb = br_ref[:, 0:1]
        rid = i * _RK + lax.broadcasted_iota(jnp.int32, (_RK, 1), 0)

        ciota = lax.broadcasted_iota(jnp.int32, (_RK, _CT), 1
                                     ).astype(jnp.float32)
        ridf = rid.astype(jnp.float32)

        def tile(ct, carry):
            bv, bi = carry
            c0 = ct * _CT
            c0f = lax.convert_element_type(c0, jnp.float32)
            cpos = posT_ref[:, pl.ds(c0, _CT)]
            csq = jnp.sum(cpos * cpos, axis=0, keepdims=True)
            dot = lax.dot_general(rpos, cpos, (((1,), (0,)), ((), ())),
                                  preferred_element_type=jnp.float32)
            dt = (rsq + csq) - 2.0 * dot
            cb = bc_ref[0:1, pl.ds(c0, _CT)]
            valid = (rb == cb) & (ridf - c0f != ciota)
            dt = jnp.where(valid, dt, _BIG)
            tv, ti = [], []
            for _ in range(_K):
                mv = jnp.min(dt, axis=1, keepdims=True)
                mil = jnp.min(jnp.where(dt == mv, ciota, _HUGE), axis=1,
                              keepdims=True)
                dt = jnp.where(ciota == mil, _HUGE, dt)
                tv.append(mv)
                ti.append(mil + c0f)
            cv = jnp.concatenate([bv[:, :_K]] + tv, axis=1)
            ci = jnp.concatenate([bi[:, :_K]] + ti, axis=1)
            nv, ni = [], []
            for _ in range(_K):
                mv = jnp.min(cv, axis=1, keepdims=True)
                mi = jnp.min(jnp.where(cv == mv, ci, _HUGE), axis=1,
                             keepdims=True)
                cv = jnp.where((cv == mv) & (ci == mi), _HUGE, cv)
                nv.append(mv)
                ni.append(mi)
            pad = jnp.full((_RK, 8 - _K), _HUGE, jnp.float32)
            return (jnp.concatenate(nv + [pad], axis=1),
                    jnp.concatenate(ni + [pad], axis=1))

        bv0 = jnp.full((_RK, 8), _HUGE, jnp.float32)
        bi0 = jnp.zeros((_RK, 8), jnp.float32)
        _, bi = lax.fori_loop(tlo_ref[i], thi_ref[i], tile, (bv0, bi0))
        out_ref[...] = jnp.minimum(bi, float(Np - 1)).astype(jnp.int32)

    return pl.pallas_call(
        body,
        grid=(nb,),
        in_specs=[
            pl.BlockSpec(memory_space=pltpu.SMEM),
            pl.BlockSpec(memory_space=pltpu.SMEM),
            pl.BlockSpec((_RK, 8), lambda i: (i, 0)),
            pl.BlockSpec((8, Np), lambda i: (0, 0)),
            pl.BlockSpec((_RK, 8), lambda i: (i, 0)),
            pl.BlockSpec((8, Np), lambda i: (0, 0)),
        ],
        out_specs=pl.BlockSpec((_RK, 8), lambda i: (i, 0)),
        out_shape=jax.ShapeDtypeStruct((Np, 8), jnp.int32),
        compiler_params=pltpu.CompilerParams(
            dimension_semantics=("arbitrary",)),
    )(tlo, thi, pos_r, posT, batch_r, batch_c)


def _sc_aggregate(h_pad, nbr_flat):
    Np, D = h_pad.shape
    rows = Np // _NW
    nch = rows // _CH
    mesh = plsc.VectorSubcoreMesh(core_axis_name="c", subcore_axis_name="s")

    @functools.partial(
        pl.kernel,
        mesh=mesh,
        out_type=jax.ShapeDtypeStruct((Np, D), jnp.float32),
        scratch_types=[
            pltpu.VMEM((_K * rows,), jnp.int32),
            pltpu.VMEM((rows, D), jnp.float32),
        ] + [pltpu.SemaphoreType.DMA] * (rows // _CH),
    )
    def k(h_hbm, idx_hbm, out_hbm, idx_v, m_v, *sems):
        wid = lax.axis_index("s") * 2 + lax.axis_index("c")
        base = wid * rows
        pltpu.sync_copy(idx_hbm.at[pl.ds(wid * (_K * rows), _K * rows)],
                        idx_v)
        k0s = [
            pltpu.async_copy(
                h_hbm.at[idx_v.at[pl.ds(c * _CH, _CH)]],
                m_v.at[pl.ds(c * _CH, _CH)], sems[c])
            for c in range(nch)
        ]
        adds = []
        for c in range(nch):
            k0s[c].wait()
            cbase = c * _CH
            dst = m_v.at[pl.ds(cbase, _CH)]
            adds.extend(
                pltpu.async_copy(
                    h_hbm.at[idx_v.at[pl.ds(kk * rows + cbase, _CH)]],
                    dst, sems[c], add=True)
                for kk in range(1, _K))
        for cp in adds:
            cp.wait()
        pltpu.sync_copy(m_v, out_hbm.at[pl.ds(base, rows)])

    return k(h_pad, nbr_flat)


def _bn_cols(y, mk, inv_n):
    ym = y * mk
    mu = jnp.sum(ym, axis=0, keepdims=True) * inv_n
    ex2 = jnp.sum(ym * y, axis=0, keepdims=True) * inv_n
    var = ex2 - mu * mu
    return mu, lax.rsqrt(var + 1e-5)


def _gin_block(h, m, e, oh, cond, w1m, w1t, g1, b1, w2, mk, inv_n):
    z = (1.0 + e) * h + m
    cw = jnp.dot(cond, w1t, preferred_element_type=jnp.float32)
    y = (jnp.dot(z, w1m, preferred_element_type=jnp.float32)
         + jnp.dot(oh, cw, preferred_element_type=jnp.float32))
    mu, rstd = _bn_cols(y, mk, inv_n)
    u = jnp.maximum((y - mu) * rstd * g1 + b1, 0.0)
    return jnp.dot(u, w2, preferred_element_type=jnp.float32)


def _dense_chain1(xp, m1, batch_r, cond_p, w1m, w1t, eps,
                  g1, b1, w2, g2, b2, n_real):
    Np, D = xp.shape
    G = cond_p.shape[0]
    inv_n = 1.0 / n_real

    def body(eps_ref, x_ref, m_ref, br_ref, c_ref, w1m_ref, w1t_ref,
             g1_ref, b1_ref, w2_ref, g2_ref, b2_ref, h1_ref):
        e = eps_ref[0, 0]
        mk = (lax.broadcasted_iota(jnp.int32, (Np, 1), 0)
              < n_real).astype(jnp.float32)
        oh = (br_ref[:, 0:1] == lax.broadcasted_iota(jnp.int32, (1, G), 1)
              ).astype(jnp.float32)
        y2 = _gin_block(x_ref[...], m_ref[...], e, oh, c_ref[...],
                        w1m_ref[...], w1t_ref[...], g1_ref[...], b1_ref[...],
                        w2_ref[...], mk, inv_n)
        mu2, rstd2 = _bn_cols(y2, mk, inv_n)
        h1_ref[...] = jnp.maximum(
            (y2 - mu2) * rstd2 * g2_ref[...] + b2_ref[...], 0.0)

    return pl.pallas_call(
        body,
        in_specs=[pl.BlockSpec(memory_space=pltpu.SMEM)] + [pl.BlockSpec()] * 11,
        out_specs=pl.BlockSpec(),
        out_shape=jax.ShapeDtypeStruct((Np, D), jnp.float32),
    )(eps, xp, m1, batch_r, cond_p, w1m, w1t, g1, b1, w2, g2, b2)


def _dense_chain2(h1, m2, batch_r, cond_p, w1m, w1t, eps,
                  g1, b1, w2, g2, b2,
                  w1p, b1p, gp, bp, w2p, b2o, n_real):
    Np, D = h1.shape
    G = cond_p.shape[0]
    inv_n = 1.0 / n_real

    def body(eps_ref, h1_ref, m_ref, br_ref, c_ref, w1m_ref, w1t_ref,
             g1_ref, b1_ref, w2_ref, g2_ref, b2_ref,
             w1p_ref, b1p_ref, gp_ref, bp_ref, w2p_ref, b2o_ref, out_ref):
        e = eps_ref[0, 0]
        mk = (lax.broadcasted_iota(jnp.int32, (Np, 1), 0)
              < n_real).astype(jnp.float32)
        oh = (br_ref[:, 0:1] == lax.broadcasted_iota(jnp.int32, (1, G), 1)
              ).astype(jnp.float32)
        h1v = h1_ref[...]
        y2 = _gin_block(h1v, m_ref[...], e, oh, c_ref[...],
                        w1m_ref[...], w1t_ref[...], g1_ref[...], b1_ref[...],
                        w2_ref[...], mk, inv_n)
        mu2, rstd2 = _bn_cols(y2, mk, inv_n)
        h2 = h1v + jnp.maximum(
            (y2 - mu2) * rstd2 * g2_ref[...] + b2_ref[...], 0.0)
        y3 = (jnp.dot(h2, w1p_ref[...], preferred_element_type=jnp.float32)
              + b1p_ref[...])
        mu3, rstd3 = _bn_cols(y3, mk, inv_n)
        o = jnp.maximum((y3 - mu3) * rstd3 * gp_ref[...] + bp_ref[...], 0.0)
        out_ref[...] = (jnp.dot(o, w2p_ref[...],
                                preferred_element_type=jnp.float32)
                        + b2o_ref[...])

    return pl.pallas_call(
        body,
        in_specs=[pl.BlockSpec(memory_space=pltpu.SMEM)] + [pl.BlockSpec()] * 17,
        out_specs=pl.BlockSpec(),
        out_shape=jax.ShapeDtypeStruct((Np, D), jnp.float32),
    )(eps, h1, m2, batch_r, cond_p, w1m, w1t, g1, b1, w2, g2, b2,
      w1p, b1p, gp, bp, w2p, b2o)


def kernel(x, batch, condition,
           gin1_eps, gin1_W1, gin1_bn1_g, gin1_bn1_b, gin1_W2, gin1_bn2_g,
           gin1_bn2_b,
           gin2_eps, gin2_W1, gin2_bn1_g, gin2_bn1_b, gin2_W2, gin2_bn2_g,
           gin2_bn2_b,
           out_W1, out_b1, out_bn_g, out_bn_b, out_W2, out_b2):
    N, Fin = x.shape
    G, C = condition.shape
    L = gin1_W2.shape[0]
    H = out_W1.shape[1]
    Fout = out_W2.shape[1]
    Np = ((N + 2047) // 2048) * 2048
    padn = Np - N

    xp = jnp.pad(x, ((0, padn), (0, 0)))
    batchp = jnp.pad(batch.astype(jnp.int32), (0, padn), constant_values=G)
    pos_r = jnp.pad(x[:, :_SPACE], ((0, padn), (0, 8 - _SPACE)))
    posT = pos_r.T
    batch_r = jnp.broadcast_to(batchp[:, None], (Np, 8))
    batch_c = jnp.broadcast_to(batchp[None, :], (8, Np))

    bl = batchp.reshape(Np // _RK, _RK)
    c_lo = jnp.searchsorted(batchp, bl[:, 0], side='left').astype(jnp.int32)
    c_hi = jnp.searchsorted(batchp, bl[:, -1], side='right').astype(jnp.int32)
    tlo = c_lo // _CT
    thi = (c_hi + _CT - 1) // _CT

    nbr8 = _knn(tlo, thi, pos_r, posT, batch_r, batch_c, Np)
    rows_w = Np // _NW
    nbr_km = (nbr8[:, :_K].reshape(_NW, rows_w, _K)
              .transpose(0, 2, 1).reshape(-1))

    cond_p = jnp.pad(condition, ((0, 0), (0, 8 - C)))
    e1 = jnp.reshape(gin1_eps, (1, 1))
    e2 = jnp.reshape(gin2_eps, (1, 1))
    w1m_a = gin1_W1[:Fin]
    w1t_a = jnp.pad(gin1_W1[Fin:], ((0, 8 - C), (0, 0)))
    w1m_b = gin2_W1[:L]
    w1t_b = jnp.pad(gin2_W1[L:], ((0, 8 - C), (0, 0)))

    def row(v):
        return jnp.reshape(v, (1, -1))

    m1 = _sc_aggregate(xp, nbr_km)
    h1 = _dense_chain1(xp, m1, batch_r, cond_p, w1m_a, w1t_a, e1,
                       row(gin1_bn1_g), row(gin1_bn1_b), gin1_W2,
                       row(gin1_bn2_g), row(gin1_bn2_b), N)

    m2 = _sc_aggregate(h1, nbr_km)
    w1p = jnp.pad(out_W1, ((0, 0), (0, L - H)))
    b1p = row(jnp.pad(out_b1, (0, L - H)))
    gp = row(jnp.pad(out_bn_g, (0, L - H)))
    bp = row(jnp.pad(out_bn_b, (0, L - H)))
    w2p = jnp.pad(out_W2, ((0, L - H), (0, 0)))
    out = _dense_chain2(h1, m2, batch_r, cond_p, w1m_b, w1t_b, e2,
                        row(gin2_bn1_g), row(gin2_bn1_b), gin2_W2,
                        row(gin2_bn2_g), row(gin2_bn2_b),
                        w1p, b1p, gp, bp, w2p, row(out_b2), N)
    return out[:N]


# speedup vs baseline: 1.1209x; 1.1209x over previous
import functools

import jax
import jax.numpy as jnp
from jax import lax
from jax.experimental import pallas as pl
from jax.experimental.pallas import tpu as pltpu
from jax.experimental.pallas import tpu_sc as plsc

_K = 5
_SPACE = 3
_BIG = float(1e10)
_HUGE = float(3e38)
_IBIG = 2**31 - 1

_RK = 256
_CT = 512
_RD = 512
_CH = 80
_NW = 32


def _knn(tlo, thi, pos_r, posT, batch_r, batch_c, Np):
    nb = Np // _RK

    def body(tlo_ref, thi_ref, posr_ref, posT_ref, br_ref, bc_ref, out_ref):
        i = pl.program_id(0)
        rpos = posr_ref[...]
        rsq = jnp.sum(rpos * rpos, axis=1, keepdims=True)
        rb = br_ref[:, 0:1]
        rid = i * _RK + lax.broadcasted_iota(jnp.int32, (_RK, 1), 0)

        ciota = lax.broadcasted_iota(jnp.int32, (_RK, _CT), 1
                                     ).astype(jnp.float32)
        ridf = rid.astype(jnp.float32)

        def tile_extract(ct):
            c0 = ct * _CT
            c0f = lax.convert_element_type(c0, jnp.float32)
            cpos = posT_ref[:, pl.ds(c0, _CT)]
            csq = jnp.sum(cpos * cpos, axis=0, keepdims=True)
            dot = lax.dot_general(rpos, cpos, (((1,), (0,)), ((), ())),
                                  preferred_element_type=jnp.float32)
            dt = (rsq + csq) - 2.0 * dot
            cb = bc_ref[0:1, pl.ds(c0, _CT)]
            valid = (rb == cb) & (ridf - c0f != ciota)
            dt = jnp.where(valid, dt, _BIG)
            tv, ti = [], []
            for _ in range(_K):
                mv = jnp.min(dt, axis=1, keepdims=True)
                mil = jnp.min(jnp.where(dt == mv, ciota, _HUGE), axis=1,
                              keepdims=True)
                dt = jnp.where(ciota == mil, _HUGE, dt)
                tv.append(mv)
                ti.append(mil + c0f)
            return tv, ti

        def tile(ct, carry):
            bv, bi = carry
            tv, ti = tile_extract(ct)
            cv = jnp.concatenate([bv[:, :_K]] + tv, axis=1)
            ci = jnp.concatenate([bi[:, :_K]] + ti, axis=1)
            nv, ni = [], []
            for _ in range(_K):
                mv = jnp.min(cv, axis=1, keepdims=True)
                mi = jnp.min(jnp.where(cv == mv, ci, _HUGE), axis=1,
                             keepdims=True)
                cv = jnp.where((cv == mv) & (ci == mi), _HUGE, cv)
                nv.append(mv)
                ni.append(mi)
            pad = jnp.full((_RK, 8 - _K), _HUGE, jnp.float32)
            return (jnp.concatenate(nv + [pad], axis=1),
                    jnp.concatenate(ni + [pad], axis=1))

        lo = tlo_ref[i]
        tv0, ti0 = tile_extract(lo)
        pad = jnp.full((_RK, 8 - _K), _HUGE, jnp.float32)
        bv0 = jnp.concatenate(tv0 + [pad], axis=1)
        bi0 = jnp.concatenate(ti0 + [pad], axis=1)
        _, bi = lax.fori_loop(lo + 1, thi_ref[i], tile, (bv0, bi0))
        out_ref[...] = jnp.minimum(bi, float(Np - 1)).astype(jnp.int32)

    return pl.pallas_call(
        body,
        grid=(nb,),
        in_specs=[
            pl.BlockSpec(memory_space=pltpu.SMEM),
            pl.BlockSpec(memory_space=pltpu.SMEM),
            pl.BlockSpec((_RK, 8), lambda i: (i, 0)),
            pl.BlockSpec((8, Np), lambda i: (0, 0)),
            pl.BlockSpec((_RK, 8), lambda i: (i, 0)),
            pl.BlockSpec((8, Np), lambda i: (0, 0)),
        ],
        out_specs=pl.BlockSpec((_RK, 8), lambda i: (i, 0)),
        out_shape=jax.ShapeDtypeStruct((Np, 8), jnp.int32),
        compiler_params=pltpu.CompilerParams(
            dimension_semantics=("arbitrary",)),
    )(tlo, thi, pos_r, posT, batch_r, batch_c)


def _sc_aggregate(h_pad, nbr_flat):
    Np, D = h_pad.shape
    rows = Np // _NW
    nch = rows // _CH
    mesh = plsc.VectorSubcoreMesh(core_axis_name="c", subcore_axis_name="s")

    @functools.partial(
        pl.kernel,
        mesh=mesh,
        out_type=jax.ShapeDtypeStruct((Np, D), jnp.float32),
        scratch_types=[
            pltpu.VMEM((_K * rows,), jnp.int32),
            pltpu.VMEM((rows, D), jnp.float32),
            pltpu.SemaphoreType.DMA,
        ],
    )
    def k(h_hbm, idx_hbm, out_hbm, idx_v, m_v, sem):
        wid = lax.axis_index("s") * 2 + lax.axis_index("c")
        base = wid * rows
        pltpu.sync_copy(idx_hbm.at[pl.ds(wid * (_K * rows), _K * rows)],
                        idx_v)

        def chunk(c, _):
            cbase = c * _CH
            dst = m_v.at[pl.ds(cbase, _CH)]
            pltpu.async_copy(
                h_hbm.at[idx_v.at[pl.ds(cbase, _CH)]], dst, sem).wait()
            cps = [
                pltpu.async_copy(
                    h_hbm.at[idx_v.at[pl.ds(kk * rows + cbase, _CH)]],
                    dst, sem, add=True)
                for kk in range(1, _K)
            ]
            for cp in cps:
                cp.wait()
            return 0

        lax.fori_loop(0, nch, chunk, 0)
        pltpu.sync_copy(m_v, out_hbm.at[pl.ds(base, rows)])

    return k(h_pad, nbr_flat)


def _bn_cols(y, mk, inv_n):
    ym = y * mk
    mu = jnp.sum(ym, axis=0, keepdims=True) * inv_n
    ex2 = jnp.sum(ym * y, axis=0, keepdims=True) * inv_n
    var = ex2 - mu * mu
    return mu, lax.rsqrt(var + 1e-5)


def _gin_block(h, m, e, oh, cond, w1m, w1t, g1, b1, w2, mk, inv_n):
    z = (1.0 + e) * h + m
    cw = jnp.dot(cond, w1t, preferred_element_type=jnp.float32)
    y = (jnp.dot(z, w1m, preferred_element_type=jnp.float32)
         + jnp.dot(oh, cw, preferred_element_type=jnp.float32))
    mu, rstd = _bn_cols(y, mk, inv_n)
    u = jnp.maximum((y - mu) * rstd * g1 + b1, 0.0)
    return jnp.dot(u, w2, preferred_element_type=jnp.float32)


def _dense_chain1(xp, m1, batch_r, cond_p, w1m, w1t, eps,
                  g1, b1, w2, g2, b2, n_real):
    Np, D = xp.shape
    G = cond_p.shape[0]
    inv_n = 1.0 / n_real

    def body(eps_ref, x_ref, m_ref, br_ref, c_ref, w1m_ref, w1t_ref,
             g1_ref, b1_ref, w2_ref, g2_ref, b2_ref, h1_ref):
        e = eps_ref[0, 0]
        mk = (lax.broadcasted_iota(jnp.int32, (Np, 1), 0)
              < n_real).astype(jnp.float32)
        oh = (br_ref[:, 0:1] == lax.broadcasted_iota(jnp.int32, (1, G), 1)
              ).astype(jnp.float32)
        y2 = _gin_block(x_ref[...], m_ref[...], e, oh, c_ref[...],
                        w1m_ref[...], w1t_ref[...], g1_ref[...], b1_ref[...],
                        w2_ref[...], mk, inv_n)
        mu2, rstd2 = _bn_cols(y2, mk, inv_n)
        h1_ref[...] = jnp.maximum(
            (y2 - mu2) * rstd2 * g2_ref[...] + b2_ref[...], 0.0)

    return pl.pallas_call(
        body,
        in_specs=[pl.BlockSpec(memory_space=pltpu.SMEM)] + [pl.BlockSpec()] * 11,
        out_specs=pl.BlockSpec(),
        out_shape=jax.ShapeDtypeStruct((Np, D), jnp.float32),
    )(eps, xp, m1, batch_r, cond_p, w1m, w1t, g1, b1, w2, g2, b2)


def _dense_chain2(h1, m2, batch_r, cond_p, w1m, w1t, eps,
                  g1, b1, w2, g2, b2,
                  w1p, b1p, gp, bp, w2p, b2o, n_real):
    Np, D = h1.shape
    G = cond_p.shape[0]
    inv_n = 1.0 / n_real

    def body(eps_ref, h1_ref, m_ref, br_ref, c_ref, w1m_ref, w1t_ref,
             g1_ref, b1_ref, w2_ref, g2_ref, b2_ref,
             w1p_ref, b1p_ref, gp_ref, bp_ref, w2p_ref, b2o_ref, out_ref):
        e = eps_ref[0, 0]
        mk = (lax.broadcasted_iota(jnp.int32, (Np, 1), 0)
              < n_real).astype(jnp.float32)
        oh = (br_ref[:, 0:1] == lax.broadcasted_iota(jnp.int32, (1, G), 1)
              ).astype(jnp.float32)
        h1v = h1_ref[...]
        y2 = _gin_block(h1v, m_ref[...], e, oh, c_ref[...],
                        w1m_ref[...], w1t_ref[...], g1_ref[...], b1_ref[...],
                        w2_ref[...], mk, inv_n)
        mu2, rstd2 = _bn_cols(y2, mk, inv_n)
        h2 = h1v + jnp.maximum(
            (y2 - mu2) * rstd2 * g2_ref[...] + b2_ref[...], 0.0)
        y3 = (jnp.dot(h2, w1p_ref[...], preferred_element_type=jnp.float32)
              + b1p_ref[...])
        mu3, rstd3 = _bn_cols(y3, mk, inv_n)
        o = jnp.maximum((y3 - mu3) * rstd3 * gp_ref[...] + bp_ref[...], 0.0)
        out_ref[...] = (jnp.dot(o, w2p_ref[...],
                                preferred_element_type=jnp.float32)
                        + b2o_ref[...])

    return pl.pallas_call(
        body,
        in_specs=[pl.BlockSpec(memory_space=pltpu.SMEM)] + [pl.BlockSpec()] * 17,
        out_specs=pl.BlockSpec(),
        out_shape=jax.ShapeDtypeStruct((Np, D), jnp.float32),
    )(eps, h1, m2, batch_r, cond_p, w1m, w1t, g1, b1, w2, g2, b2,
      w1p, b1p, gp, bp, w2p, b2o)


def kernel(x, batch, condition,
           gin1_eps, gin1_W1, gin1_bn1_g, gin1_bn1_b, gin1_W2, gin1_bn2_g,
           gin1_bn2_b,
           gin2_eps, gin2_W1, gin2_bn1_g, gin2_bn1_b, gin2_W2, gin2_bn2_g,
           gin2_bn2_b,
           out_W1, out_b1, out_bn_g, out_bn_b, out_W2, out_b2):
    N, Fin = x.shape
    G, C = condition.shape
    L = gin1_W2.shape[0]
    H = out_W1.shape[1]
    Fout = out_W2.shape[1]
    Np = ((N + 2047) // 2048) * 2048
    padn = Np - N

    xp = jnp.pad(x, ((0, padn), (0, 0)))
    batchp = jnp.pad(batch.astype(jnp.int32), (0, padn), constant_values=G)
    pos_r = jnp.pad(x[:, :_SPACE], ((0, padn), (0, 8 - _SPACE)))
    posT = pos_r.T
    batch_r = jnp.broadcast_to(batchp[:, None], (Np, 8))
    batch_c = jnp.broadcast_to(batchp[None, :], (8, Np))

    bl = batchp.reshape(Np // _RK, _RK)
    c_lo = jnp.searchsorted(batchp, bl[:, 0], side='left').astype(jnp.int32)
    c_hi = jnp.searchsorted(batchp, bl[:, -1], side='right').astype(jnp.int32)
    tlo = c_lo // _CT
    thi = (c_hi + _CT - 1) // _CT

    nbr8 = _knn(tlo, thi, pos_r, posT, batch_r, batch_c, Np)
    rows_w = Np // _NW
    nbr_km = (nbr8[:, :_K].reshape(_NW, rows_w, _K)
              .transpose(0, 2, 1).reshape(-1))

    cond_p = jnp.pad(condition, ((0, 0), (0, 8 - C)))
    e1 = jnp.reshape(gin1_eps, (1, 1))
    e2 = jnp.reshape(gin2_eps, (1, 1))
    w1m_a = gin1_W1[:Fin]
    w1t_a = jnp.pad(gin1_W1[Fin:], ((0, 8 - C), (0, 0)))
    w1m_b = gin2_W1[:L]
    w1t_b = jnp.pad(gin2_W1[L:], ((0, 8 - C), (0, 0)))

    def row(v):
        return jnp.reshape(v, (1, -1))

    m1 = _sc_aggregate(xp, nbr_km)
    h1 = _dense_chain1(xp, m1, batch_r, cond_p, w1m_a, w1t_a, e1,
                       row(gin1_bn1_g), row(gin1_bn1_b), gin1_W2,
                       row(gin1_bn2_g), row(gin1_bn2_b), N)

    m2 = _sc_aggregate(h1, nbr_km)
    w1p = jnp.pad(out_W1, ((0, 0), (0, L - H)))
    b1p = row(jnp.pad(out_b1, (0, L - H)))
    gp = row(jnp.pad(out_bn_g, (0, L - H)))
    bp = row(jnp.pad(out_bn_b, (0, L - H)))
    w2p = jnp.pad(out_W2, ((0, L - H), (0, 0)))
    out = _dense_chain2(h1, m2, batch_r, cond_p, w1m_b, w1t_b, e2,
                        row(gin2_bn1_g), row(gin2_bn1_b), gin2_W2,
                        row(gin2_bn2_g), row(gin2_bn2_b),
                        w1p, b1p, gp, bp, w2p, row(out_b2), N)
    return out[:N]
